# baseline (speedup 1.0000x reference)
import functools

import jax
import jax.numpy as jnp
from jax import lax
from jax.experimental import pallas as pl
from jax.experimental.pallas import tpu as pltpu
from jax.experimental.pallas import tpu_sc as plsc

_K = 36
_V = 768
_NW = 32
_R = 64


def _sc_counts(xflat, B):
    rows_per_w = B // _NW
    n_super = rows_per_w // (2 * _R)
    xw_words = rows_per_w * _K
    cw_words = _R * _V
    mesh = plsc.VectorSubcoreMesh(core_axis_name="c", subcore_axis_name="s")

    @functools.partial(
        pl.kernel,
        mesh=mesh,
        out_type=jax.ShapeDtypeStruct((B * _V,), jnp.float32),
        compiler_params=pltpu.CompilerParams(needs_layout_passes=False),
        scratch_types=[
            pltpu.VMEM((xw_words,), jnp.int32),
            pltpu.VMEM((cw_words,), jnp.float32),
            pltpu.VMEM((cw_words,), jnp.float32),
            pltpu.SemaphoreType.DMA,
            pltpu.SemaphoreType.DMA,
        ],
    )
    def k(x_hbm, c_hbm, x_v, c_v0, c_v1, sem0, sem1):
        wid = lax.axis_index("s") * 2 + lax.axis_index("c")
        lanes = lax.iota(jnp.int32, 16)
        zeros = jnp.zeros((16,), jnp.float32)

        pltpu.sync_copy(x_hbm.at[pl.ds(wid * xw_words, xw_words)], x_v)

        def zbody(j, carry):
            for u in range(8):
                off = (j * 8 + u) * 16
                c_v0[pl.ds(off, 16)] = zeros
                c_v1[pl.ds(off, 16)] = zeros
            return carry

        lax.fori_loop(0, cw_words // (16 * 8), zbody, 0)

        def scatter_chunk(ch, buf, sign):
            vals = jnp.full((16,), sign, jnp.float32)

            def gbody(g, carry):
                rows_local = g * 16 + lanes
                xbase = (ch * _R + rows_local) * _K
                cbase = rows_local * _V
                for i in range(_K):
                    idx = plsc.load_gather(x_v, [xbase + i])
                    valid = idx < _V
                    plsc.addupdate_scatter(buf, [cbase + idx], vals, mask=valid)
                return carry

            lax.fori_loop(0, _R // 16, gbody, 0)

        def out_dma(ch, buf, sem):
            row0 = wid * rows_per_w + ch * _R
            return pltpu.async_copy(
                buf, c_hbm.at[pl.ds(row0 * _V, cw_words)], sem)

        def super_body(t, carry):
            ch0 = 2 * t
            ch1 = 2 * t + 1
            scatter_chunk(ch0, c_v0, 1.0)
            d0 = out_dma(ch0, c_v0, sem0)
            scatter_chunk(ch1, c_v1, 1.0)
            d1 = out_dma(ch1, c_v1, sem1)
            d0.wait()
            scatter_chunk(ch0, c_v0, -1.0)
            d1.wait()
            scatter_chunk(ch1, c_v1, -1.0)
            return carry

        lax.fori_loop(0, n_super, super_body, 0)

    return k(xflat)


def _tc_reduce(C, T_aug, turn, scal, B):
    BM = 2048

    def body(c_ref, t_ref, u_ref, s_ref, o_ref):
        Cb = c_ref[...]
        P = jnp.dot(Cb.astype(jnp.bfloat16), t_ref[...],
                    preferred_element_type=jnp.float32)
        q = jnp.sum(Cb * P[:, :_V], axis=1, keepdims=True)
        n = jnp.sum(Cb, axis=1, keepdims=True)
        c3 = Cb[:, 3:4]
        c443 = Cb[:, 443:444]
        p3 = P[:, 3:4]
        p443 = P[:, 443:444]
        p768 = P[:, _V:_V + 1]
        p769 = P[:, _V + 1:_V + 2]
        t33 = s_ref[0, 0]
        t443 = s_ref[0, 1]
        t440_3 = s_ref[0, 2]
        outw = q - c3 * (p3 + p768) + c3 * c3 * t33
        outb = (q - c443 * (p443 + p769) + c443 * c443 * t443
                + (float(_K * _K) - n * n) * t440_3)
        o_ref[...] = jnp.where(u_ref[...] == 1, outw, outb)

    return pl.pallas_call(
        body,
        grid=(B // BM,),
        in_specs=[
            pl.BlockSpec((BM, _V), lambda i: (i, 0)),
            pl.BlockSpec((_V, _V + 128), lambda i: (0, 0)),
            pl.BlockSpec((BM, 1), lambda i: (i, 0)),
            pl.BlockSpec((1, 128), lambda i: (0, 0)),
        ],
        out_specs=pl.BlockSpec((BM, 1), lambda i: (i, 0)),
        out_shape=jax.ShapeDtypeStruct((B, 1), jnp.float32),
    )(C, T_aug, turn, scal)


def kernel(x, turn, tiles, zeros_param):
    B = x.shape[0]
    x32 = x.astype(jnp.int32).reshape(B * _K)
    C = _sc_counts(x32, B).reshape(B, _V)

    T2 = tiles.reshape(_V, _V)
    T_aug = jnp.concatenate(
        [T2, T2[3:4, :].T, T2[443:444, :].T,
         jnp.zeros((_V, 126), jnp.float32)], axis=1).astype(jnp.bfloat16)
    scal = jnp.zeros((1, 128), jnp.float32)
    scal = scal.at[0, 0].set(T2[3, 3]).at[0, 1].set(T2[443, 443])
    scal = scal.at[0, 2].set(T2[440, 3])

    out = _tc_reduce(C, T_aug, turn.astype(jnp.int32), scal, B)
    return (out, jnp.zeros((1,), dtype=out.dtype))


# speedup vs baseline: 1.2700x; 1.2700x over previous
import functools

import jax
import jax.numpy as jnp
from jax import lax
from jax.experimental import pallas as pl
from jax.experimental.pallas import tpu as pltpu
from jax.experimental.pallas import tpu_sc as plsc

_K = 36
_V = 768
_NW = 32
_R = 64


def _sc_counts(xflat, B):
    rows_per_w = B // _NW
    n_super = rows_per_w // (2 * _R)
    xw_words = rows_per_w * _K
    mesh = plsc.VectorSubcoreMesh(core_axis_name="c", subcore_axis_name="s")

    @functools.partial(
        pl.kernel,
        mesh=mesh,
        out_type=jax.ShapeDtypeStruct((B, _V), jnp.float32),
        compiler_params=pltpu.CompilerParams(needs_layout_passes=False),
        scratch_types=[
            pltpu.VMEM((xw_words,), jnp.int32),
            pltpu.VMEM((_R, _V), jnp.float32),
            pltpu.VMEM((_R, _V), jnp.float32),
            pltpu.SemaphoreType.DMA,
            pltpu.SemaphoreType.DMA,
        ],
    )
    def k(x_hbm, c_hbm, x_v, c_v0, c_v1, sem0, sem1):
        wid = lax.axis_index("s") * 2 + lax.axis_index("c")
        lanes = lax.iota(jnp.int32, 16)
        zeros = jnp.zeros((16,), jnp.float32)

        pltpu.sync_copy(x_hbm.at[pl.ds(wid * xw_words, xw_words)], x_v)

        def zbody(j, carry):
            for u in range(8):
                off = (j * 8 + u) * 16
                r = off // _V
                c = off % _V
                c_v0[r, pl.ds(c, 16)] = zeros
                c_v1[r, pl.ds(c, 16)] = zeros
            return carry

        lax.fori_loop(0, _R * _V // (16 * 8), zbody, 0)

        def scatter_chunk(ch, buf, sign):
            vals = jnp.full((16,), sign, jnp.float32)

            def gbody(g, carry):
                rows_local = g * 16 + lanes
                xbase = (ch * _R + rows_local) * _K
                for i in range(_K):
                    idx = plsc.load_gather(x_v, [xbase + i])
                    valid = idx < _V
                    plsc.addupdate_scatter(
                        buf, [rows_local, idx], vals, mask=valid)
                return carry

            lax.fori_loop(0, _R // 16, gbody, 0)

        def out_dma(ch, buf, sem):
            row0 = wid * rows_per_w + ch * _R
            return pltpu.async_copy(buf, c_hbm.at[pl.ds(row0, _R)], sem)

        def super_body(t, carry):
            ch0 = 2 * t
            ch1 = 2 * t + 1
            scatter_chunk(ch0, c_v0, 1.0)
            d0 = out_dma(ch0, c_v0, sem0)
            scatter_chunk(ch1, c_v1, 1.0)
            d1 = out_dma(ch1, c_v1, sem1)
            d0.wait()
            scatter_chunk(ch0, c_v0, -1.0)
            d1.wait()
            scatter_chunk(ch1, c_v1, -1.0)
            return carry

        lax.fori_loop(0, n_super, super_body, 0)

    return k(xflat)


def _tc_reduce(C, T_aug, turn, scal, B):
    BM = 2048

    def body(c_ref, t_ref, u_ref, s_ref, o_ref):
        Cb = c_ref[...]
        P = jnp.dot(Cb.astype(jnp.bfloat16), t_ref[...],
                    preferred_element_type=jnp.float32)
        q = jnp.sum(Cb * P[:, :_V], axis=1, keepdims=True)
        n = jnp.sum(Cb, axis=1, keepdims=True)
        c3 = Cb[:, 3:4]
        c443 = Cb[:, 443:444]
        p3 = P[:, 3:4]
        p443 = P[:, 443:444]
        p768 = P[:, _V:_V + 1]
        p769 = P[:, _V + 1:_V + 2]
        t33 = s_ref[0, 0]
        t443 = s_ref[0, 1]
        t440_3 = s_ref[0, 2]
        outw = q - c3 * (p3 + p768) + c3 * c3 * t33
        outb = (q - c443 * (p443 + p769) + c443 * c443 * t443
                + (float(_K * _K) - n * n) * t440_3)
        o_ref[...] = jnp.where(u_ref[...] == 1, outw, outb)

    return pl.pallas_call(
        body,
        grid=(B // BM,),
        in_specs=[
            pl.BlockSpec((BM, _V), lambda i: (i, 0)),
            pl.BlockSpec((_V, _V + 128), lambda i: (0, 0)),
            pl.BlockSpec((BM, 1), lambda i: (i, 0)),
            pl.BlockSpec((1, 128), lambda i: (0, 0)),
        ],
        out_specs=pl.BlockSpec((BM, 1), lambda i: (i, 0)),
        out_shape=jax.ShapeDtypeStruct((B, 1), jnp.float32),
    )(C, T_aug, turn, scal)


def kernel(x, turn, tiles, zeros_param):
    B = x.shape[0]
    x32 = x.astype(jnp.int32).reshape(B * _K)
    C = _sc_counts(x32, B)

    T2 = tiles.reshape(_V, _V)
    T_aug = jnp.concatenate(
        [T2, T2[3:4, :].T, T2[443:444, :].T,
         jnp.zeros((_V, 126), jnp.float32)], axis=1).astype(jnp.bfloat16)
    scal = jnp.pad(
        jnp.stack([T2[3, 3], T2[443, 443], T2[440, 3]])[None, :],
        ((0, 0), (0, 125)))

    out = _tc_reduce(C, T_aug, turn.astype(jnp.int32), scal, B)
    return (out, jnp.zeros((1,), dtype=out.dtype))


# speedup vs baseline: 2.0023x; 1.5766x over previous
import functools

import jax
import jax.numpy as jnp
from jax import lax
from jax.experimental import pallas as pl
from jax.experimental.pallas import tpu as pltpu
from jax.experimental.pallas import tpu_sc as plsc

_K = 36
_V = 768
_NW = 32
_R = 64


def _sc_counts(xflat, B):
    rows_per_w = B // _NW
    n_super = rows_per_w // (2 * _R)
    xw_words = rows_per_w * _K
    mesh = plsc.VectorSubcoreMesh(core_axis_name="c", subcore_axis_name="s")

    @functools.partial(
        pl.kernel,
        mesh=mesh,
        out_type=jax.ShapeDtypeStruct((B, _V), jnp.float32),
        compiler_params=pltpu.CompilerParams(needs_layout_passes=False),
        scratch_types=[
            pltpu.VMEM((xw_words,), jnp.int32),
            pltpu.VMEM((_R, _V), jnp.float32),
            pltpu.VMEM((_R, _V), jnp.float32),
            pltpu.SemaphoreType.DMA,
            pltpu.SemaphoreType.DMA,
        ],
    )
    def k(x_hbm, c_hbm, x_v, c_v0, c_v1, sem0, sem1):
        wid = lax.axis_index("s") * 2 + lax.axis_index("c")
        lanes = lax.iota(jnp.int32, 16)
        zeros = jnp.zeros((16,), jnp.float32)

        pltpu.sync_copy(x_hbm.at[pl.ds(wid * xw_words, xw_words)], x_v)

        def zbody(j, carry):
            for u in range(8):
                off = (j * 8 + u) * 16
                r = off // _V
                c = off % _V
                c_v0[r, pl.ds(c, 16)] = zeros
                c_v1[r, pl.ds(c, 16)] = zeros
            return carry

        lax.fori_loop(0, _R * _V // (16 * 8), zbody, 0)

        def scatter_chunk(ch, buf, sign):
            vals = jnp.full((16,), sign, jnp.float32)

            def gbody(g, carry):
                rows_local = g * 16 + lanes
                xbase = (ch * _R + rows_local) * _K
                for i in range(_K):
                    idx = plsc.load_gather(x_v, [xbase + i])
                    valid = idx < _V
                    plsc.addupdate_scatter(
                        buf, [rows_local, idx], vals, mask=valid)
                return carry

            lax.fori_loop(0, _R // 16, gbody, 0)

        def out_dma(ch, buf, sem):
            row0 = wid * rows_per_w + ch * _R
            return pltpu.async_copy(buf, c_hbm.at[pl.ds(row0, _R)], sem)

        def super_body(t, carry):
            ch0 = 2 * t
            ch1 = 2 * t + 1
            scatter_chunk(ch0, c_v0, 1.0)
            d0 = out_dma(ch0, c_v0, sem0)
            scatter_chunk(ch1, c_v1, 1.0)
            d1 = out_dma(ch1, c_v1, sem1)
            d0.wait()
            scatter_chunk(ch0, c_v0, -1.0)
            d1.wait()
            scatter_chunk(ch1, c_v1, -1.0)
            return carry

        lax.fori_loop(0, n_super, super_body, 0)

    return k(xflat)


def _tc_repack(t3):
    RM = 96

    def body(t_ref, o_ref):
        t = t_ref[...].reshape(RM, 64, 12)
        t = jnp.swapaxes(t, 1, 2)
        o_ref[...] = t.reshape(RM, _V).astype(jnp.bfloat16)

    return pl.pallas_call(
        body,
        grid=(_V // RM,),
        in_specs=[pl.BlockSpec((RM * 64, 12), lambda i: (i, 0))],
        out_specs=pl.BlockSpec((RM, _V), lambda i: (i, 0)),
        out_shape=jax.ShapeDtypeStruct((_V, _V), jnp.bfloat16),
    )(t3)


def _tc_reduce(C, T2b, extra, turn, scal, B):
    BM = 2048

    def body(c_ref, t_ref, e_ref, u_ref, s_ref, o_ref):
        Cb = c_ref[...]
        Cb16 = Cb.astype(jnp.bfloat16)
        P = jnp.dot(Cb16, t_ref[...], preferred_element_type=jnp.float32)
        PE = jnp.dot(Cb16, e_ref[...], preferred_element_type=jnp.float32)
        q = jnp.sum(Cb * P, axis=1, keepdims=True)
        n = jnp.sum(Cb, axis=1, keepdims=True)
        c3 = Cb[:, 3:4]
        c443 = Cb[:, 443:444]
        p3 = P[:, 3:4]
        p443 = P[:, 443:444]
        p768 = PE[:, 0:1]
        p769 = PE[:, 1:2]
        t33 = s_ref[0, 0]
        t443 = s_ref[0, 1]
        t440_3 = s_ref[0, 2]
        outw = q - c3 * (p3 + p768) + c3 * c3 * t33
        outb = (q - c443 * (p443 + p769) + c443 * c443 * t443
                + (float(_K * _K) - n * n) * t440_3)
        o_ref[...] = jnp.where(u_ref[...] == 1, outw, outb)

    return pl.pallas_call(
        body,
        grid=(B // BM,),
        in_specs=[
            pl.BlockSpec((BM, _V), lambda i: (i, 0)),
            pl.BlockSpec((_V, _V), lambda i: (0, 0)),
            pl.BlockSpec((_V, 128), lambda i: (0, 0)),
            pl.BlockSpec((BM, 1), lambda i: (i, 0)),
            pl.BlockSpec((1, 128), lambda i: (0, 0)),
        ],
        out_specs=pl.BlockSpec((BM, 1), lambda i: (i, 0)),
        out_shape=jax.ShapeDtypeStruct((B, 1), jnp.float32),
    )(C, T2b, extra, turn, scal)


def kernel(x, turn, tiles, zeros_param):
    B = x.shape[0]
    x32 = x.astype(jnp.int32).reshape(B * _K)
    C = _sc_counts(x32, B)

    t3 = jnp.transpose(tiles, (0, 1, 2, 4, 5, 6, 3)).reshape(49152, 12)
    T2b = _tc_repack(t3)

    extra = jnp.concatenate(
        [T2b[3:4], T2b[443:444], jnp.zeros((126, _V), jnp.bfloat16)],
        axis=0).T
    scal = jnp.pad(
        jnp.stack([T2b[3, 3], T2b[443, 443], T2b[440, 3]]).astype(
            jnp.float32)[None, :],
        ((0, 0), (0, 125)))

    out = _tc_reduce(C, T2b, extra, turn.astype(jnp.int32), scal, B)
    return (out, jnp.zeros((1,), dtype=out.dtype))


# speedup vs baseline: 2.0705x; 1.0341x over previous
import functools

import jax
import jax.numpy as jnp
from jax import lax
from jax.experimental import pallas as pl
from jax.experimental.pallas import tpu as pltpu
from jax.experimental.pallas import tpu_sc as plsc

_K = 36
_V = 768
_NW = 32
_R = 64


def _sc_counts(xflat, B):
    rows_per_w = B // _NW
    n_super = rows_per_w // (2 * _R)
    xw_words = rows_per_w * _K
    mesh = plsc.VectorSubcoreMesh(core_axis_name="c", subcore_axis_name="s")

    @functools.partial(
        pl.kernel,
        mesh=mesh,
        out_type=jax.ShapeDtypeStruct((B, _V), jnp.float32),
        compiler_params=pltpu.CompilerParams(needs_layout_passes=False),
        scratch_types=[
            pltpu.VMEM((xw_words,), jnp.int32),
            pltpu.VMEM((_R, _V), jnp.float32),
            pltpu.VMEM((_R, _V), jnp.float32),
            pltpu.SemaphoreType.DMA,
            pltpu.SemaphoreType.DMA,
        ],
    )
    def k(x_hbm, c_hbm, x_v, c_v0, c_v1, sem0, sem1):
        wid = lax.axis_index("s") * 2 + lax.axis_index("c")
        lanes = lax.iota(jnp.int32, 16)
        zeros = jnp.zeros((16,), jnp.float32)

        pltpu.sync_copy(x_hbm.at[pl.ds(wid * xw_words, xw_words)], x_v)

        def zbody(j, carry):
            for u in range(8):
                off = (j * 8 + u) * 16
                r = off // _V
                c = off % _V
                c_v0[r, pl.ds(c, 16)] = zeros
                c_v1[r, pl.ds(c, 16)] = zeros
            return carry

        lax.fori_loop(0, _R * _V // (16 * 8), zbody, 0)

        def scatter_chunk(ch, buf, sign):
            vals = jnp.full((16,), sign, jnp.float32)

            def gbody(g, carry):
                rows_local = g * 16 + lanes
                xbase = (ch * _R + rows_local) * _K
                for i in range(_K):
                    idx = plsc.load_gather(x_v, [xbase + i])
                    valid = idx < _V
                    plsc.addupdate_scatter(
                        buf, [rows_local, idx], vals, mask=valid)
                return carry

            lax.fori_loop(0, _R // 16, gbody, 0)

        def out_dma(ch, buf, sem):
            row0 = wid * rows_per_w + ch * _R
            return pltpu.async_copy(buf, c_hbm.at[pl.ds(row0, _R)], sem)

        def super_body(t, carry):
            ch0 = 2 * t
            ch1 = 2 * t + 1
            scatter_chunk(ch0, c_v0, 1.0)
            d0 = out_dma(ch0, c_v0, sem0)
            scatter_chunk(ch1, c_v1, 1.0)
            d1 = out_dma(ch1, c_v1, sem1)
            d0.wait()
            scatter_chunk(ch0, c_v0, -1.0)
            d1.wait()
            scatter_chunk(ch1, c_v1, -1.0)
            return carry

        lax.fori_loop(0, n_super, super_body, 0)

    return k(xflat)


def _tc_repack(t3):
    RM = 96

    def body(t_ref, o_ref):
        t = t_ref[...].reshape(RM, 64, 12)
        t = jnp.swapaxes(t, 1, 2)
        o_ref[...] = t.reshape(RM, _V).astype(jnp.bfloat16)

    return pl.pallas_call(
        body,
        grid=(_V // RM,),
        in_specs=[pl.BlockSpec((RM * 64, 12), lambda i: (i, 0))],
        out_specs=pl.BlockSpec((RM, _V), lambda i: (i, 0)),
        out_shape=jax.ShapeDtypeStruct((_V, _V), jnp.bfloat16),
    )(t3)


def _tc_reduce(C, Sb, turn, scal, B):
    BM = 4096

    def body(c_ref, t_ref, u_ref, s_ref, o_ref):
        Cb = c_ref[...]
        Cb16 = Cb.astype(jnp.bfloat16)
        P = jnp.dot(Cb16, t_ref[...], preferred_element_type=jnp.float32)
        q = jnp.sum(Cb * P, axis=1, keepdims=True)
        n = jnp.sum(Cb, axis=1, keepdims=True)
        c3 = Cb[:, 3:4]
        c443 = Cb[:, 443:444]
        p3 = P[:, 3:4]
        p443 = P[:, 443:444]
        t33 = s_ref[0, 0]
        t443 = s_ref[0, 1]
        t440_3 = s_ref[0, 2]
        outw = q - 2.0 * c3 * p3 + c3 * c3 * t33
        outb = (q - 2.0 * c443 * p443 + c443 * c443 * t443
                + (float(_K * _K) - n * n) * t440_3)
        o_ref[...] = jnp.where(u_ref[...] == 1, outw, outb)

    return pl.pallas_call(
        body,
        grid=(B // BM,),
        in_specs=[
            pl.BlockSpec((BM, _V), lambda i: (i, 0)),
            pl.BlockSpec((_V, _V), lambda i: (0, 0)),
            pl.BlockSpec((BM, 1), lambda i: (i, 0)),
            pl.BlockSpec((1, 128), lambda i: (0, 0)),
        ],
        out_specs=pl.BlockSpec((BM, 1), lambda i: (i, 0)),
        out_shape=jax.ShapeDtypeStruct((B, 1), jnp.float32),
    )(C, Sb, turn, scal)


def kernel(x, turn, tiles, zeros_param):
    B = x.shape[0]
    x32 = x.astype(jnp.int32).reshape(B * _K)
    C = _sc_counts(x32, B)

    t3 = jnp.transpose(tiles, (0, 1, 2, 4, 5, 6, 3)).reshape(49152, 12)
    T2b = _tc_repack(t3)

    Sb = ((T2b + T2b.T) * jnp.bfloat16(0.5))
    scal = jnp.pad(
        jnp.stack([T2b[3, 3], T2b[443, 443], T2b[440, 3]]).astype(
            jnp.float32)[None, :],
        ((0, 0), (0, 125)))

    out = _tc_reduce(C, Sb, turn.astype(jnp.int32), scal, B)
    return (out, jnp.zeros((1,), dtype=out.dtype))


# speedup vs baseline: 2.3126x; 1.1170x over previous
import functools

import jax
import jax.numpy as jnp
from jax import lax
from jax.experimental import pallas as pl
from jax.experimental.pallas import tpu as pltpu
from jax.experimental.pallas import tpu_sc as plsc

_K = 36
_V = 768
_NW = 32
_R = 64


def _sc_counts(xT, B):
    rows_per_w = B // _NW
    n_super = rows_per_w // (2 * _R)
    mesh = plsc.VectorSubcoreMesh(core_axis_name="c", subcore_axis_name="s")

    @functools.partial(
        pl.kernel,
        mesh=mesh,
        out_type=jax.ShapeDtypeStruct((B, _V), jnp.float32),
        compiler_params=pltpu.CompilerParams(needs_layout_passes=False),
        scratch_types=[
            pltpu.VMEM((_K, rows_per_w), jnp.int32),
            pltpu.VMEM((_R, _V), jnp.float32),
            pltpu.VMEM((_R, _V), jnp.float32),
            pltpu.SemaphoreType.DMA,
            pltpu.SemaphoreType.DMA,
        ],
    )
    def k(x_hbm, c_hbm, x_v, c_v0, c_v1, sem0, sem1):
        wid = lax.axis_index("s") * 2 + lax.axis_index("c")
        lanes = lax.iota(jnp.int32, 16)
        zeros = jnp.zeros((16,), jnp.float32)

        pltpu.sync_copy(x_hbm.at[:, pl.ds(wid * rows_per_w, rows_per_w)], x_v)

        def zbody(j, carry):
            for u in range(8):
                off = (j * 8 + u) * 16
                r = off // _V
                c = off % _V
                c_v0[r, pl.ds(c, 16)] = zeros
                c_v1[r, pl.ds(c, 16)] = zeros
            return carry

        lax.fori_loop(0, _R * _V // (16 * 8), zbody, 0)

        def scatter_chunk(ch, buf, sign):
            vals = jnp.full((16,), sign, jnp.float32)

            def gbody(g, carry):
                rows_local = g * 16 + lanes
                rows_w = ch * _R + rows_local
                for i in range(_K):
                    idx = plsc.load_gather(
                        x_v, [jnp.full((16,), i, jnp.int32), rows_w])
                    valid = idx < _V
                    plsc.addupdate_scatter(
                        buf, [rows_local, idx], vals, mask=valid)
                return carry

            lax.fori_loop(0, _R // 16, gbody, 0)

        def out_dma(ch, buf, sem):
            row0 = wid * rows_per_w + ch * _R
            return pltpu.async_copy(buf, c_hbm.at[pl.ds(row0, _R)], sem)

        def super_body(t, carry):
            ch0 = 2 * t
            ch1 = 2 * t + 1
            scatter_chunk(ch0, c_v0, 1.0)
            d0 = out_dma(ch0, c_v0, sem0)
            scatter_chunk(ch1, c_v1, 1.0)
            d1 = out_dma(ch1, c_v1, sem1)
            d0.wait()
            scatter_chunk(ch0, c_v0, -1.0)
            d1.wait()
            scatter_chunk(ch1, c_v1, -1.0)
            return carry

        lax.fori_loop(0, n_super, super_body, 0)

    return k(xT)


def _tc_repack(t3):
    RM = 96

    def body(t_ref, o_ref):
        t = t_ref[...].reshape(RM, 64, 12)
        t = jnp.swapaxes(t, 1, 2)
        o_ref[...] = t.reshape(RM, _V).astype(jnp.bfloat16)

    return pl.pallas_call(
        body,
        grid=(_V // RM,),
        in_specs=[pl.BlockSpec((RM * 64, 12), lambda i: (i, 0))],
        out_specs=pl.BlockSpec((RM, _V), lambda i: (i, 0)),
        out_shape=jax.ShapeDtypeStruct((_V, _V), jnp.bfloat16),
    )(t3)


def _tc_reduce(C, Sb, turn, scal, B):
    BM = 4096

    def body(c_ref, t_ref, u_ref, s_ref, o_ref):
        Cb = c_ref[...]
        Cb16 = Cb.astype(jnp.bfloat16)
        P = jnp.dot(Cb16, t_ref[...], preferred_element_type=jnp.float32)
        q = jnp.sum(Cb * P, axis=1, keepdims=True)
        n = jnp.sum(Cb, axis=1, keepdims=True)
        c3 = Cb[:, 3:4]
        c443 = Cb[:, 443:444]
        p3 = P[:, 3:4]
        p443 = P[:, 443:444]
        t33 = s_ref[0, 0]
        t443 = s_ref[0, 1]
        t440_3 = s_ref[0, 2]
        outw = q - 2.0 * c3 * p3 + c3 * c3 * t33
        outb = (q - 2.0 * c443 * p443 + c443 * c443 * t443
                + (float(_K * _K) - n * n) * t440_3)
        o_ref[...] = jnp.where(u_ref[...] == 1, outw, outb)

    return pl.pallas_call(
        body,
        grid=(B // BM,),
        in_specs=[
            pl.BlockSpec((BM, _V), lambda i: (i, 0)),
            pl.BlockSpec((_V, _V), lambda i: (0, 0)),
            pl.BlockSpec((BM, 1), lambda i: (i, 0)),
            pl.BlockSpec((1, 128), lambda i: (0, 0)),
        ],
        out_specs=pl.BlockSpec((BM, 1), lambda i: (i, 0)),
        out_shape=jax.ShapeDtypeStruct((B, 1), jnp.float32),
    )(C, Sb, turn, scal)


def kernel(x, turn, tiles, zeros_param):
    B = x.shape[0]
    xT = x.astype(jnp.int32).T
    C = _sc_counts(xT, B)

    t3 = jnp.transpose(tiles, (0, 1, 2, 4, 5, 6, 3)).reshape(49152, 12)
    T2b = _tc_repack(t3)

    Sb = ((T2b + T2b.T) * jnp.bfloat16(0.5))
    scal = jnp.pad(
        jnp.stack([T2b[3, 3], T2b[443, 443], T2b[440, 3]]).astype(
            jnp.float32)[None, :],
        ((0, 0), (0, 125)))

    out = _tc_reduce(C, Sb, turn.astype(jnp.int32), scal, B)
    return (out, jnp.zeros((1,), dtype=out.dtype))


# speedup vs baseline: 2.4666x; 1.0666x over previous
import functools

import jax
import jax.numpy as jnp
from jax import lax
from jax.experimental import pallas as pl
from jax.experimental.pallas import tpu as pltpu
from jax.experimental.pallas import tpu_sc as plsc

_K = 36
_V = 768
_NW = 32
_R = 64


def _sc_counts(xT, B):
    rows_per_w = B // _NW
    prows_per_w = rows_per_w // 2
    n_super = prows_per_w // (2 * _R)
    mesh = plsc.VectorSubcoreMesh(core_axis_name="c", subcore_axis_name="s")

    @functools.partial(
        pl.kernel,
        mesh=mesh,
        out_type=jax.ShapeDtypeStruct((B // 2, _V), jnp.int32),
        compiler_params=pltpu.CompilerParams(needs_layout_passes=False),
        scratch_types=[
            pltpu.VMEM((_K, rows_per_w), jnp.int32),
            pltpu.VMEM((_R, _V), jnp.int32),
            pltpu.VMEM((_R, _V), jnp.int32),
            pltpu.SemaphoreType.DMA,
            pltpu.SemaphoreType.DMA,
        ],
    )
    def k(x_hbm, c_hbm, x_v, c_v0, c_v1, sem0, sem1):
        wid = lax.axis_index("s") * 2 + lax.axis_index("c")
        lanes = lax.iota(jnp.int32, 16)
        zeros = jnp.zeros((16,), jnp.int32)

        pltpu.sync_copy(x_hbm.at[:, pl.ds(wid * rows_per_w, rows_per_w)], x_v)

        def zbody(j, carry):
            for u in range(8):
                off = (j * 8 + u) * 16
                r = off // _V
                c = off % _V
                c_v0[r, pl.ds(c, 16)] = zeros
                c_v1[r, pl.ds(c, 16)] = zeros
            return carry

        lax.fori_loop(0, _R * _V // (16 * 8), zbody, 0)

        def scatter_chunk(ch, buf, sign):
            v_lo = jnp.full((16,), sign, jnp.int32)
            v_hi = jnp.full((16,), sign * 65536, jnp.int32)

            def gbody(g, carry):
                rp_local = g * 16 + lanes
                rp_w = ch * _R + rp_local
                for i in range(_K):
                    icol = jnp.full((16,), i, jnp.int32)
                    idx_lo = plsc.load_gather(x_v, [icol, 2 * rp_w])
                    plsc.addupdate_scatter(
                        buf, [rp_local, idx_lo], v_lo, mask=idx_lo < _V)
                    idx_hi = plsc.load_gather(x_v, [icol, 2 * rp_w + 1])
                    plsc.addupdate_scatter(
                        buf, [rp_local, idx_hi], v_hi, mask=idx_hi < _V)
                return carry

            lax.fori_loop(0, _R // 16, gbody, 0)

        def out_dma(ch, buf, sem):
            row0 = wid * prows_per_w + ch * _R
            return pltpu.async_copy(buf, c_hbm.at[pl.ds(row0, _R)], sem)

        def super_body(t, carry):
            ch0 = 2 * t
            ch1 = 2 * t + 1
            scatter_chunk(ch0, c_v0, 1)
            d0 = out_dma(ch0, c_v0, sem0)
            scatter_chunk(ch1, c_v1, 1)
            d1 = out_dma(ch1, c_v1, sem1)
            d0.wait()
            scatter_chunk(ch0, c_v0, -1)
            d1.wait()
            scatter_chunk(ch1, c_v1, -1)
            return carry

        lax.fori_loop(0, n_super, super_body, 0)

    return k(xT)


def _tc_repack(t3):
    RM = 96

    def body(t_ref, o_ref):
        t = t_ref[...].reshape(RM, 64, 12)
        t = jnp.swapaxes(t, 1, 2)
        o_ref[...] = t.reshape(RM, _V).astype(jnp.bfloat16)

    return pl.pallas_call(
        body,
        grid=(_V // RM,),
        in_specs=[pl.BlockSpec((RM * 64, 12), lambda i: (i, 0))],
        out_specs=pl.BlockSpec((RM, _V), lambda i: (i, 0)),
        out_shape=jax.ShapeDtypeStruct((_V, _V), jnp.bfloat16),
    )(t3)


def _tc_reduce(Cp, Sb, turn_lo, turn_hi, scal, B):
    H = B // 2
    BM = 2048

    def stream_out(Cs, P, u, s_ref):
        q = jnp.sum(Cs * P, axis=1, keepdims=True)
        n = jnp.sum(Cs, axis=1, keepdims=True)
        c3 = Cs[:, 3:4]
        c443 = Cs[:, 443:444]
        p3 = P[:, 3:4]
        p443 = P[:, 443:444]
        t33 = s_ref[0, 0]
        t443 = s_ref[0, 1]
        t440_3 = s_ref[0, 2]
        outw = q - 2.0 * c3 * p3 + c3 * c3 * t33
        outb = (q - 2.0 * c443 * p443 + c443 * c443 * t443
                + (float(_K * _K) - n * n) * t440_3)
        return jnp.where(u == 1, outw, outb)

    def body(c_ref, t_ref, ul_ref, uh_ref, s_ref, ol_ref, oh_ref):
        w = c_ref[...]
        Clo = jnp.bitwise_and(w, 0xFFFF).astype(jnp.float32)
        Chi = jnp.right_shift(w, 16).astype(jnp.float32)
        Sbv = t_ref[...]
        P_lo = jnp.dot(Clo.astype(jnp.bfloat16), Sbv,
                       preferred_element_type=jnp.float32)
        P_hi = jnp.dot(Chi.astype(jnp.bfloat16), Sbv,
                       preferred_element_type=jnp.float32)
        ol_ref[...] = stream_out(Clo, P_lo, ul_ref[...], s_ref)
        oh_ref[...] = stream_out(Chi, P_hi, uh_ref[...], s_ref)

    return pl.pallas_call(
        body,
        grid=(H // BM,),
        in_specs=[
            pl.BlockSpec((BM, _V), lambda i: (i, 0)),
            pl.BlockSpec((_V, _V), lambda i: (0, 0)),
            pl.BlockSpec((BM, 1), lambda i: (i, 0)),
            pl.BlockSpec((BM, 1), lambda i: (i, 0)),
            pl.BlockSpec((1, 128), lambda i: (0, 0)),
        ],
        out_specs=[
            pl.BlockSpec((BM, 1), lambda i: (i, 0)),
            pl.BlockSpec((BM, 1), lambda i: (i, 0)),
        ],
        out_shape=[
            jax.ShapeDtypeStruct((H, 1), jnp.float32),
            jax.ShapeDtypeStruct((H, 1), jnp.float32),
        ],
    )(Cp, Sb, turn_lo, turn_hi, scal)


def kernel(x, turn, tiles, zeros_param):
    B = x.shape[0]
    xT = x.astype(jnp.int32).T
    Cp = _sc_counts(xT, B)

    t3 = jnp.transpose(tiles, (0, 1, 2, 4, 5, 6, 3)).reshape(49152, 12)
    T2b = _tc_repack(t3)

    Sb = ((T2b + T2b.T) * jnp.bfloat16(0.5))
    scal = jnp.pad(
        jnp.stack([T2b[3, 3], T2b[443, 443], T2b[440, 3]]).astype(
            jnp.float32)[None, :],
        ((0, 0), (0, 125)))

    turn32 = turn.astype(jnp.int32)
    out_lo, out_hi = _tc_reduce(
        Cp, Sb, turn32[0::2], turn32[1::2], scal, B)
    out = jnp.concatenate([out_lo, out_hi], axis=1).reshape(B, 1)
    return (out, jnp.zeros((1,), dtype=out.dtype))


# speedup vs baseline: 2.6097x; 1.0580x over previous
import functools

import jax
import jax.numpy as jnp
from jax import lax
from jax.experimental import pallas as pl
from jax.experimental.pallas import tpu as pltpu
from jax.experimental.pallas import tpu_sc as plsc

_K = 36
_V = 768
_NW = 32
_R = 64


def _sc_counts(xT, B):
    rows_per_w = B // _NW
    prows_per_w = rows_per_w // 2
    n_super = prows_per_w // (2 * _R)
    mesh = plsc.VectorSubcoreMesh(core_axis_name="c", subcore_axis_name="s")

    @functools.partial(
        pl.kernel,
        mesh=mesh,
        out_type=jax.ShapeDtypeStruct((B // 2, _V), jnp.int32),
        compiler_params=pltpu.CompilerParams(needs_layout_passes=False),
        scratch_types=[
            pltpu.VMEM((_K, rows_per_w), jnp.int32),
            pltpu.VMEM((_R, _V), jnp.int32),
            pltpu.VMEM((_R, _V), jnp.int32),
            pltpu.SemaphoreType.DMA,
            pltpu.SemaphoreType.DMA,
        ],
    )
    def k(x_hbm, c_hbm, x_v, c_v0, c_v1, sem0, sem1):
        wid = lax.axis_index("s") * 2 + lax.axis_index("c")
        lanes = lax.iota(jnp.int32, 16)
        zeros = jnp.zeros((16,), jnp.int32)

        pltpu.sync_copy(x_hbm.at[:, pl.ds(wid * rows_per_w, rows_per_w)], x_v)

        def zbody(j, carry):
            for u in range(8):
                off = (j * 8 + u) * 16
                r = off // _V
                c = off % _V
                c_v0[r, pl.ds(c, 16)] = zeros
                c_v1[r, pl.ds(c, 16)] = zeros
            return carry

        lax.fori_loop(0, _R * _V // (16 * 8), zbody, 0)

        def scatter_chunk(ch, buf, sign):
            v_lo = jnp.full((16,), sign, jnp.int32)
            v_hi = jnp.full((16,), sign * 65536, jnp.int32)

            def gbody(g, carry):
                rp_local = g * 16 + lanes
                rp_w = ch * _R + rp_local
                for i in range(_K):
                    icol = jnp.full((16,), i, jnp.int32)
                    idx_lo = plsc.load_gather(x_v, [icol, 2 * rp_w])
                    plsc.addupdate_scatter(
                        buf, [rp_local, idx_lo], v_lo, mask=idx_lo < _V)
                    idx_hi = plsc.load_gather(x_v, [icol, 2 * rp_w + 1])
                    plsc.addupdate_scatter(
                        buf, [rp_local, idx_hi], v_hi, mask=idx_hi < _V)
                return carry

            lax.fori_loop(0, _R // 16, gbody, 0)

        def out_dma(ch, buf, sem):
            row0 = wid * prows_per_w + ch * _R
            return pltpu.async_copy(buf, c_hbm.at[pl.ds(row0, _R)], sem)

        def super_body(t, carry):
            ch0 = 2 * t
            ch1 = 2 * t + 1
            scatter_chunk(ch0, c_v0, 1)
            d0 = out_dma(ch0, c_v0, sem0)
            scatter_chunk(ch1, c_v1, 1)
            d1 = out_dma(ch1, c_v1, sem1)
            d0.wait()
            scatter_chunk(ch0, c_v0, -1)
            d1.wait()
            scatter_chunk(ch1, c_v1, -1)
            return carry

        lax.fori_loop(0, n_super, super_body, 0)

    return k(xT)


def _tc_repack(t3):
    RM = 96

    def body(t_ref, o_ref):
        t = t_ref[...].reshape(RM, 64, 12)
        t = jnp.swapaxes(t, 1, 2)
        o_ref[...] = t.reshape(RM, _V).astype(jnp.bfloat16)

    return pl.pallas_call(
        body,
        grid=(_V // RM,),
        in_specs=[pl.BlockSpec((RM * 64, 12), lambda i: (i, 0))],
        out_specs=pl.BlockSpec((RM, _V), lambda i: (i, 0)),
        out_shape=jax.ShapeDtypeStruct((_V, _V), jnp.bfloat16),
    )(t3)


def _tc_reduce(Cp, Sb, turn_lo, turn_hi, scal, B):
    H = B // 2
    BM = 1024

    def stream_out(Cs, P, u, s_ref):
        q = jnp.sum(Cs * P, axis=1, keepdims=True)
        n = jnp.sum(Cs, axis=1, keepdims=True)
        c3 = Cs[:, 3:4]
        c443 = Cs[:, 443:444]
        p3 = P[:, 3:4]
        p443 = P[:, 443:444]
        t33 = s_ref[0, 0]
        t443 = s_ref[0, 1]
        t440_3 = s_ref[0, 2]
        outw = q - 2.0 * c3 * p3 + c3 * c3 * t33
        outb = (q - 2.0 * c443 * p443 + c443 * c443 * t443
                + (float(_K * _K) - n * n) * t440_3)
        return jnp.where(u == 1, outw, outb)

    def body(c_ref, t_ref, ul_ref, uh_ref, s_ref, ol_ref, oh_ref):
        w = c_ref[...]
        Clo = jnp.bitwise_and(w, 0xFFFF).astype(jnp.float32)
        Chi = jnp.right_shift(w, 16).astype(jnp.float32)
        Sbv = t_ref[...]
        P_lo = jnp.dot(Clo.astype(jnp.bfloat16), Sbv,
                       preferred_element_type=jnp.float32)
        P_hi = jnp.dot(Chi.astype(jnp.bfloat16), Sbv,
                       preferred_element_type=jnp.float32)
        ol_ref[...] = stream_out(Clo, P_lo, ul_ref[...], s_ref)
        oh_ref[...] = stream_out(Chi, P_hi, uh_ref[...], s_ref)

    return pl.pallas_call(
        body,
        grid=(H // BM,),
        in_specs=[
            pl.BlockSpec((BM, _V), lambda i: (i, 0)),
            pl.BlockSpec((_V, _V), lambda i: (0, 0)),
            pl.BlockSpec((BM, 1), lambda i: (i, 0)),
            pl.BlockSpec((BM, 1), lambda i: (i, 0)),
            pl.BlockSpec((1, 128), lambda i: (0, 0)),
        ],
        out_specs=[
            pl.BlockSpec((BM, 1), lambda i: (i, 0)),
            pl.BlockSpec((BM, 1), lambda i: (i, 0)),
        ],
        out_shape=[
            jax.ShapeDtypeStruct((H, 1), jnp.float32),
            jax.ShapeDtypeStruct((H, 1), jnp.float32),
        ],
    )(Cp, Sb, turn_lo, turn_hi, scal)


def kernel(x, turn, tiles, zeros_param):
    B = x.shape[0]
    xT = x.astype(jnp.int32).T
    Cp = _sc_counts(xT, B)

    t3 = jnp.transpose(tiles, (0, 1, 2, 4, 5, 6, 3)).reshape(49152, 12)
    T2b = _tc_repack(t3)

    Sb = ((T2b + T2b.T) * jnp.bfloat16(0.5))
    scal = jnp.pad(
        jnp.stack([T2b[3, 3], T2b[443, 443], T2b[440, 3]]).astype(
            jnp.float32)[None, :],
        ((0, 0), (0, 125)))

    turn32 = turn.astype(jnp.int32)
    out_lo, out_hi = _tc_reduce(
        Cp, Sb, turn32[0::2], turn32[1::2], scal, B)
    out = jnp.concatenate([out_lo, out_hi], axis=1).reshape(B, 1)
    return (out, jnp.zeros((1,), dtype=out.dtype))


# speedup vs baseline: 2.7220x; 1.0430x over previous
import functools

import jax
import jax.numpy as jnp
from jax import lax
from jax.experimental import pallas as pl
from jax.experimental.pallas import tpu as pltpu
from jax.experimental.pallas import tpu_sc as plsc

_K = 36
_V = 768
_NW = 32
_R = 64


def _sc_counts(xT, B):
    rows_per_w = B // _NW
    prows_per_w = rows_per_w // 2
    n_super = prows_per_w // (2 * _R)
    mesh = plsc.VectorSubcoreMesh(core_axis_name="c", subcore_axis_name="s")

    @functools.partial(
        pl.kernel,
        mesh=mesh,
        out_type=jax.ShapeDtypeStruct((B // 2, _V), jnp.int32),
        compiler_params=pltpu.CompilerParams(needs_layout_passes=False),
        scratch_types=[
            pltpu.VMEM((_K, rows_per_w), jnp.int32),
            pltpu.VMEM((_R, _V), jnp.int32),
            pltpu.VMEM((_R, _V), jnp.int32),
            pltpu.SemaphoreType.DMA,
            pltpu.SemaphoreType.DMA,
        ],
    )
    def k(x_hbm, c_hbm, x_v, c_v0, c_v1, sem0, sem1):
        wid = lax.axis_index("s") * 2 + lax.axis_index("c")
        lanes = lax.iota(jnp.int32, 16)
        zeros = jnp.zeros((16,), jnp.int32)

        pltpu.sync_copy(x_hbm.at[:, pl.ds(wid * rows_per_w, rows_per_w)], x_v)

        def zbody(j, carry):
            for u in range(8):
                off = (j * 8 + u) * 16
                r = off // _V
                c = off % _V
                c_v0[r, pl.ds(c, 16)] = zeros
                c_v1[r, pl.ds(c, 16)] = zeros
            return carry

        lax.fori_loop(0, _R * _V // (16 * 8), zbody, 0)

        def scatter_chunk(ch, buf, sign):
            v_lo = jnp.full((16,), sign, jnp.int32)
            v_hi = jnp.full((16,), sign * 65536, jnp.int32)

            def gbody(g, carry):
                rp_local = g * 16 + lanes
                rp_w = ch * _R + rp_local
                for i in range(_K):
                    icol = jnp.full((16,), i, jnp.int32)
                    idx_lo = plsc.load_gather(x_v, [icol, 2 * rp_w])
                    plsc.addupdate_scatter(
                        buf, [rp_local, idx_lo], v_lo, mask=idx_lo < _V)
                    idx_hi = plsc.load_gather(x_v, [icol, 2 * rp_w + 1])
                    plsc.addupdate_scatter(
                        buf, [rp_local, idx_hi], v_hi, mask=idx_hi < _V)
                return carry

            lax.fori_loop(0, _R // 16, gbody, 0)

        def out_dma(ch, buf, sem):
            row0 = wid * prows_per_w + ch * _R
            return pltpu.async_copy(buf, c_hbm.at[pl.ds(row0, _R)], sem)

        def super_body(t, carry):
            ch0 = 2 * t
            ch1 = 2 * t + 1
            scatter_chunk(ch0, c_v0, 1)
            d0 = out_dma(ch0, c_v0, sem0)
            scatter_chunk(ch1, c_v1, 1)
            d1 = out_dma(ch1, c_v1, sem1)
            d0.wait()
            scatter_chunk(ch0, c_v0, -1)
            d1.wait()
            scatter_chunk(ch1, c_v1, -1)
            return carry

        lax.fori_loop(0, n_super, super_body, 0)

    return k(xT)


def _tc_repack(t3):
    RM = 96

    def body(t_ref, o_ref):
        t = t_ref[...].reshape(RM, 64, 12)
        t = jnp.swapaxes(t, 1, 2)
        o_ref[...] = t.reshape(RM, _V).astype(jnp.bfloat16)

    return pl.pallas_call(
        body,
        grid=(_V // RM,),
        in_specs=[pl.BlockSpec((RM * 64, 12), lambda i: (i, 0))],
        out_specs=pl.BlockSpec((RM, _V), lambda i: (i, 0)),
        out_shape=jax.ShapeDtypeStruct((_V, _V), jnp.bfloat16),
    )(t3)


def _tc_reduce(Cp, Sb, turn_lo, turn_hi, scal, B):
    H = B // 2
    BM = 512

    def stream_out(Cs, P, u, s_ref):
        q = jnp.sum(Cs * P, axis=1, keepdims=True)
        n = jnp.sum(Cs, axis=1, keepdims=True)
        c3 = Cs[:, 3:4]
        c443 = Cs[:, 443:444]
        p3 = P[:, 3:4]
        p443 = P[:, 443:444]
        t33 = s_ref[0, 0]
        t443 = s_ref[0, 1]
        t440_3 = s_ref[0, 2]
        outw = q - 2.0 * c3 * p3 + c3 * c3 * t33
        outb = (q - 2.0 * c443 * p443 + c443 * c443 * t443
                + (float(_K * _K) - n * n) * t440_3)
        return jnp.where(u == 1, outw, outb)

    def body(c_ref, t_ref, ul_ref, uh_ref, s_ref, ol_ref, oh_ref):
        w = c_ref[...]
        Clo = jnp.bitwise_and(w, 0xFFFF).astype(jnp.float32)
        Chi = jnp.right_shift(w, 16).astype(jnp.float32)
        Sbv = t_ref[...]
        P_lo = jnp.dot(Clo.astype(jnp.bfloat16), Sbv,
                       preferred_element_type=jnp.float32)
        P_hi = jnp.dot(Chi.astype(jnp.bfloat16), Sbv,
                       preferred_element_type=jnp.float32)
        ol_ref[...] = stream_out(Clo, P_lo, ul_ref[...], s_ref)
        oh_ref[...] = stream_out(Chi, P_hi, uh_ref[...], s_ref)

    return pl.pallas_call(
        body,
        grid=(H // BM,),
        in_specs=[
            pl.BlockSpec((BM, _V), lambda i: (i, 0)),
            pl.BlockSpec((_V, _V), lambda i: (0, 0)),
            pl.BlockSpec((BM, 1), lambda i: (i, 0)),
            pl.BlockSpec((BM, 1), lambda i: (i, 0)),
            pl.BlockSpec((1, 128), lambda i: (0, 0)),
        ],
        out_specs=[
            pl.BlockSpec((BM, 1), lambda i: (i, 0)),
            pl.BlockSpec((BM, 1), lambda i: (i, 0)),
        ],
        out_shape=[
            jax.ShapeDtypeStruct((H, 1), jnp.float32),
            jax.ShapeDtypeStruct((H, 1), jnp.float32),
        ],
    )(Cp, Sb, turn_lo, turn_hi, scal)


def kernel(x, turn, tiles, zeros_param):
    B = x.shape[0]
    xT = x.astype(jnp.int32).T
    Cp = _sc_counts(xT, B)

    t3 = jnp.transpose(tiles, (0, 1, 2, 4, 5, 6, 3)).reshape(49152, 12)
    T2b = _tc_repack(t3)

    Sb = ((T2b + T2b.T) * jnp.bfloat16(0.5))
    scal = jnp.pad(
        jnp.stack([T2b[3, 3], T2b[443, 443], T2b[440, 3]]).astype(
            jnp.float32)[None, :],
        ((0, 0), (0, 125)))

    turn32 = turn.astype(jnp.int32)
    out_lo, out_hi = _tc_reduce(
        Cp, Sb, turn32[0::2], turn32[1::2], scal, B)
    out = jnp.concatenate([out_lo, out_hi], axis=1).reshape(B, 1)
    return (out, jnp.zeros((1,), dtype=out.dtype))
